# baseline (speedup 1.0000x reference)
import jax
import jax.numpy as jnp
from jax import lax
from jax.experimental import pallas as pl
from jax.experimental.pallas import tpu as pltpu

_S = 1024


def _mask_body(rem_ref, x_ref, o_ref):
    cols = lax.broadcasted_iota(jnp.int32, (1, 17), 1) + 1
    keep = jnp.ones((1, 17), dtype=jnp.bool_)
    for j in range(rem_ref.shape[0]):
        keep = jnp.logical_and(keep, cols != rem_ref[j])
    m = keep.astype(o_ref.dtype).reshape(1, 1, 1, 17)
    o_ref[...] = x_ref[...] * m


def kernel(x, removed_electrodes):
    B, C, T, E = x.shape
    grid = (B, T // _S)
    return pl.pallas_call(
        _mask_body,
        grid=grid,
        in_specs=[
            pl.BlockSpec(memory_space=pltpu.SMEM),
            pl.BlockSpec((1, C, _S, E), lambda i, j: (i, 0, j, 0)),
        ],
        out_specs=pl.BlockSpec((1, C, _S, E), lambda i, j: (i, 0, j, 0)),
        out_shape=jax.ShapeDtypeStruct(x.shape, x.dtype),
    )(removed_electrodes.astype(jnp.int32), x)


# speedup vs baseline: 2.9892x; 2.9892x over previous
import jax
import jax.numpy as jnp
from jax.experimental import pallas as pl
from jax.experimental.pallas import tpu as pltpu


def _mask_body(rem_ref, x_ref, o_ref):
    e = pl.program_id(1)
    keep = jnp.int32(1)
    for j in range(rem_ref.shape[0]):
        keep = keep * (e + 1 != rem_ref[j]).astype(jnp.int32)
    o_ref[...] = x_ref[...] * keep.astype(o_ref.dtype)


def kernel(x, removed_electrodes):
    B, C, T, E = x.shape
    xt = jnp.transpose(x, (0, 3, 1, 2))
    out_t = pl.pallas_call(
        _mask_body,
        grid=(B, E),
        in_specs=[
            pl.BlockSpec(memory_space=pltpu.SMEM),
            pl.BlockSpec((1, 1, C, T), lambda i, j: (i, j, 0, 0)),
        ],
        out_specs=pl.BlockSpec((1, 1, C, T), lambda i, j: (i, j, 0, 0)),
        out_shape=jax.ShapeDtypeStruct((B, E, C, T), x.dtype),
    )(removed_electrodes.astype(jnp.int32), xt)
    return jnp.transpose(out_t, (0, 2, 3, 1))


# speedup vs baseline: 18.5411x; 6.2026x over previous
import jax
import jax.numpy as jnp
from jax.experimental import pallas as pl
from jax.experimental.pallas import tpu as pltpu


def _mask_body(rem_ref, x_ref, o_ref):
    E = x_ref.shape[1]
    for e in range(E):
        keep = jnp.int32(1)
        for j in range(rem_ref.shape[0]):
            keep = keep * (e + 1 != rem_ref[j]).astype(jnp.int32)
        o_ref[0, e] = x_ref[0, e] * keep.astype(o_ref.dtype)


def kernel(x, removed_electrodes):
    B, C, T, E = x.shape
    xt = jnp.transpose(x, (0, 3, 1, 2))
    out_t = pl.pallas_call(
        _mask_body,
        grid=(B,),
        in_specs=[
            pl.BlockSpec(memory_space=pltpu.SMEM),
            pl.BlockSpec((1, E, C, T), lambda i: (i, 0, 0, 0)),
        ],
        out_specs=pl.BlockSpec((1, E, C, T), lambda i: (i, 0, 0, 0)),
        out_shape=jax.ShapeDtypeStruct((B, E, C, T), x.dtype),
    )(removed_electrodes.astype(jnp.int32), xt)
    return jnp.transpose(out_t, (0, 2, 3, 1))


# speedup vs baseline: 20.0099x; 1.0792x over previous
import jax
import jax.numpy as jnp
from jax.experimental import pallas as pl
from jax.experimental.pallas import tpu as pltpu


_BB = 2


def _mask_body(rem_ref, x_ref, o_ref):
    E = x_ref.shape[1]
    for b in range(x_ref.shape[0]):
        for e in range(E):
            keep = jnp.int32(1)
            for j in range(rem_ref.shape[0]):
                keep = keep * (e + 1 != rem_ref[j]).astype(jnp.int32)
            o_ref[b, e] = x_ref[b, e] * keep.astype(o_ref.dtype)


def kernel(x, removed_electrodes):
    B, C, T, E = x.shape
    xt = jnp.transpose(x, (0, 3, 1, 2))
    out_t = pl.pallas_call(
        _mask_body,
        grid=(B // _BB,),
        in_specs=[
            pl.BlockSpec(memory_space=pltpu.SMEM),
            pl.BlockSpec((_BB, E, C, T), lambda i: (i, 0, 0, 0)),
        ],
        out_specs=pl.BlockSpec((_BB, E, C, T), lambda i: (i, 0, 0, 0)),
        out_shape=jax.ShapeDtypeStruct((B, E, C, T), x.dtype),
    )(removed_electrodes.astype(jnp.int32), xt)
    return jnp.transpose(out_t, (0, 2, 3, 1))
